# baseline (speedup 1.0000x reference)
import functools

import jax
import jax.numpy as jnp
from jax import lax
from jax.experimental import pallas as pl
from jax.experimental.pallas import tpu as pltpu
from jax.experimental.pallas import tpu_sc as plsc

N_CLASSES = 1000000
EMBED_DIM = 64
BATCH = 16384

_INFO = plsc.get_sparse_core_info()
_NC = _INFO.num_cores
_NS = _INFO.num_subcores
_NW = _NC * _NS
_B_PER_W = BATCH // _NW
_CHUNK = 128
_NCHUNKS = _B_PER_W // _CHUNK


@functools.partial(
    pl.kernel,
    mesh=plsc.VectorSubcoreMesh(core_axis_name="c", subcore_axis_name="s"),
    out_type=jax.ShapeDtypeStruct((BATCH, EMBED_DIM), jnp.float32),
    scratch_types=[
        pltpu.VMEM((_NCHUNKS, _CHUNK), jnp.int32),
        pltpu.VMEM((_B_PER_W,), jnp.int32),
        pltpu.VMEM((2, _CHUNK, 2 * EMBED_DIM), jnp.float32),
        pltpu.VMEM((_B_PER_W, EMBED_DIM), jnp.float32),
        pltpu.SemaphoreType.DMA,
        pltpu.SemaphoreType.DMA,
    ],
    compiler_params=pltpu.CompilerParams(needs_layout_passes=False),
)
def _embed_lookup(idx_hbm, half_hbm, table2_hbm, out_hbm, idx_v, half_v,
                  pairs_v, out_v, sem0, sem1):
    wid = lax.axis_index("s") * _NC + lax.axis_index("c")
    base = wid * _B_PER_W
    pltpu.sync_copy(idx_hbm.at[wid], idx_v)
    pltpu.sync_copy(half_hbm.at[pl.ds(base, _B_PER_W)], half_v)
    sems = (sem0, sem1)
    lanes = lax.iota(jnp.int32, 16)

    def fire(j):
        return pltpu.async_copy(
            table2_hbm.at[idx_v.at[j]], pairs_v.at[j % 2], sems[j % 2]
        )

    pending = fire(0)
    for j in range(_NCHUNKS):
        pending.wait()
        if j + 1 < _NCHUNKS:
            pending = fire(j + 1)
        bufv = lanes * 0 + (j % 2)
        k_base = j * _CHUNK

        def select(g):
            kvec = lanes + g * 16
            col0 = half_v[pl.ds(k_base + g * 16, 16)] * EMBED_DIM
            kvec_abs = kvec + k_base
            for c in range(EMBED_DIM):
                vals = plsc.load_gather(pairs_v, [bufv, kvec, col0 + c])
                plsc.store_scatter(out_v, [kvec_abs, lanes * 0 + c], vals)

        pl.loop(0, _CHUNK // 16)(select)

    pltpu.sync_copy(out_v, out_hbm.at[pl.ds(base, _B_PER_W)])


def kernel(class_ids, table):
    idx = class_ids.astype(jnp.int32)
    pair_idx = (idx // 2).reshape(_NW, _NCHUNKS, _CHUNK)
    half = idx & 1
    out = _embed_lookup(pair_idx, half, table.reshape(500000, 2 * EMBED_DIM))
    return out.reshape(BATCH, 1, EMBED_DIM)


# speedup vs baseline: 1.1670x; 1.1670x over previous
import functools

import jax
import jax.numpy as jnp
from jax import lax
from jax.experimental import pallas as pl
from jax.experimental.pallas import tpu as pltpu
from jax.experimental.pallas import tpu_sc as plsc

N_CLASSES = 1000000
EMBED_DIM = 64
BATCH = 16384

_INFO = plsc.get_sparse_core_info()
_NC = _INFO.num_cores
_NS = _INFO.num_subcores
_NW = _NC * _NS
_B_PER_W = BATCH // _NW
_CHUNK = 128
_NCHUNKS = _B_PER_W // _CHUNK


@functools.partial(
    pl.kernel,
    mesh=plsc.VectorSubcoreMesh(core_axis_name="c", subcore_axis_name="s"),
    out_type=jax.ShapeDtypeStruct((BATCH, 2 * EMBED_DIM), jnp.float32),
    scratch_types=[
        pltpu.VMEM((_NCHUNKS, _CHUNK), jnp.int32),
        pltpu.VMEM((_B_PER_W, 2 * EMBED_DIM), jnp.float32),
        pltpu.SemaphoreType.DMA,
    ],
)
def _embed_lookup(idx_hbm, tablep_hbm, out_hbm, idx_v, rows_v, sem):
    wid = lax.axis_index("s") * _NC + lax.axis_index("c")
    base = wid * _B_PER_W
    pltpu.sync_copy(idx_hbm.at[wid], idx_v)
    copies = []
    for j in range(_NCHUNKS):
        copies.append(
            pltpu.async_copy(
                tablep_hbm.at[idx_v.at[j]],
                rows_v.at[pl.ds(j * _CHUNK, _CHUNK)],
                sem,
            )
        )
    for c in copies:
        c.wait()
    pltpu.sync_copy(rows_v, out_hbm.at[pl.ds(base, _B_PER_W)])


def kernel(class_ids, table):
    idx = class_ids.astype(jnp.int32).reshape(_NW, _NCHUNKS, _CHUNK)
    table_p = jnp.pad(table, ((0, 0), (0, EMBED_DIM)))
    out = _embed_lookup(idx, table_p)
    return out[:, :EMBED_DIM].reshape(BATCH, 1, EMBED_DIM)


# speedup vs baseline: 1.3295x; 1.1393x over previous
import functools

import jax
import jax.numpy as jnp
from jax import lax
from jax.experimental import pallas as pl
from jax.experimental.pallas import tpu as pltpu
from jax.experimental.pallas import tpu_sc as plsc

N_CLASSES = 1000000
EMBED_DIM = 64
BATCH = 16384

_INFO = plsc.get_sparse_core_info()
_NC = _INFO.num_cores
_NS = _INFO.num_subcores
_NW = _NC * _NS
_B_PER_W = BATCH // _NW
_CHUNK = 128
_NCHUNKS = _B_PER_W // _CHUNK

_BC = 2048
_GRID = (N_CLASSES + _BC - 1) // _BC


def _transpose_pad_body(tt_ref, out_ref):
    blk = tt_ref[...]
    out_ref[...] = jnp.pad(blk.T, ((0, 0), (0, EMBED_DIM)))


_transpose_pad = pl.pallas_call(
    _transpose_pad_body,
    grid=(_GRID,),
    in_specs=[pl.BlockSpec((EMBED_DIM, _BC), lambda i: (0, i))],
    out_specs=pl.BlockSpec((_BC, 2 * EMBED_DIM), lambda i: (i, 0)),
    out_shape=jax.ShapeDtypeStruct((N_CLASSES, 2 * EMBED_DIM), jnp.float32),
)


@functools.partial(
    pl.kernel,
    mesh=plsc.VectorSubcoreMesh(core_axis_name="c", subcore_axis_name="s"),
    out_type=jax.ShapeDtypeStruct((BATCH, 2 * EMBED_DIM), jnp.float32),
    scratch_types=[
        pltpu.VMEM((_NCHUNKS, _CHUNK), jnp.int32),
        pltpu.VMEM((_B_PER_W, 2 * EMBED_DIM), jnp.float32),
        pltpu.SemaphoreType.DMA,
    ],
)
def _embed_lookup(idx_hbm, tablep_hbm, out_hbm, idx_v, rows_v, sem):
    wid = lax.axis_index("s") * _NC + lax.axis_index("c")
    base = wid * _B_PER_W
    pltpu.sync_copy(idx_hbm.at[wid], idx_v)
    copies = []
    for j in range(_NCHUNKS):
        copies.append(
            pltpu.async_copy(
                tablep_hbm.at[idx_v.at[j]],
                rows_v.at[pl.ds(j * _CHUNK, _CHUNK)],
                sem,
            )
        )
    for c in copies:
        c.wait()
    pltpu.sync_copy(rows_v, out_hbm.at[pl.ds(base, _B_PER_W)])


def kernel(class_ids, table):
    idx = class_ids.astype(jnp.int32).reshape(_NW, _NCHUNKS, _CHUNK)
    table_p = _transpose_pad(table.T)
    out = _embed_lookup(idx, table_p)
    return out[:, :EMBED_DIM].reshape(BATCH, 1, EMBED_DIM)


# speedup vs baseline: 2.2172x; 1.6677x over previous
import functools

import jax
import jax.numpy as jnp
from jax import lax
from jax.experimental import pallas as pl
from jax.experimental.pallas import tpu as pltpu
from jax.experimental.pallas import tpu_sc as plsc

N_CLASSES = 1000000
EMBED_DIM = 64
BATCH = 16384

_INFO = plsc.get_sparse_core_info()
_NC = _INFO.num_cores
_NS = _INFO.num_subcores
_NW = _NC * _NS
_B_PER_W = BATCH // _NW
_CHUNK = 128
_NCHUNKS = _B_PER_W // _CHUNK

_BC = 8192
_GRID = (N_CLASSES + _BC - 1) // _BC


def _transpose_pad_body(tt_ref, out_ref):
    blk = tt_ref[...]
    out_ref[...] = jnp.pad(blk.T, ((0, 0), (0, EMBED_DIM)))


_transpose_pad = pl.pallas_call(
    _transpose_pad_body,
    grid=(_GRID,),
    in_specs=[pl.BlockSpec((EMBED_DIM, _BC), lambda i: (0, i))],
    out_specs=pl.BlockSpec((_BC, 2 * EMBED_DIM), lambda i: (i, 0)),
    out_shape=jax.ShapeDtypeStruct((N_CLASSES, 2 * EMBED_DIM), jnp.float32),
)


@functools.partial(
    pl.kernel,
    mesh=plsc.VectorSubcoreMesh(core_axis_name="c", subcore_axis_name="s"),
    out_type=jax.ShapeDtypeStruct((BATCH, 2 * EMBED_DIM), jnp.float32),
    scratch_types=[
        pltpu.VMEM((_NCHUNKS, _CHUNK), jnp.int32),
        pltpu.VMEM((_B_PER_W, 2 * EMBED_DIM), jnp.float32),
        pltpu.SemaphoreType.DMA,
    ],
)
def _embed_lookup(idx_hbm, tablep_hbm, out_hbm, idx_v, rows_v, sem):
    wid = lax.axis_index("s") * _NC + lax.axis_index("c")
    base = wid * _B_PER_W
    pltpu.sync_copy(idx_hbm.at[wid], idx_v)
    copies = []
    for j in range(_NCHUNKS):
        copies.append(
            pltpu.async_copy(
                tablep_hbm.at[idx_v.at[j]],
                rows_v.at[pl.ds(j * _CHUNK, _CHUNK)],
                sem,
            )
        )
    for c in copies:
        c.wait()
    pltpu.sync_copy(rows_v, out_hbm.at[pl.ds(base, _B_PER_W)])


def kernel(class_ids, table):
    idx = class_ids.astype(jnp.int32).reshape(_NW, _NCHUNKS, _CHUNK)
    table_p = _transpose_pad(table.T)
    out = _embed_lookup(idx, table_p)
    return out[:, :EMBED_DIM].reshape(BATCH, 1, EMBED_DIM)


# speedup vs baseline: 2.4174x; 1.0903x over previous
import functools

import jax
import jax.numpy as jnp
from jax import lax
from jax.experimental import pallas as pl
from jax.experimental.pallas import tpu as pltpu
from jax.experimental.pallas import tpu_sc as plsc

N_CLASSES = 1000000
EMBED_DIM = 64
BATCH = 16384

_INFO = plsc.get_sparse_core_info()
_NC = _INFO.num_cores
_NS = _INFO.num_subcores
_NW = _NC * _NS
_B_PER_W = BATCH // _NW
_CHUNK = 128
_NCHUNKS = _B_PER_W // _CHUNK

_BC = 32768
_GRID = (N_CLASSES + _BC - 1) // _BC


def _transpose_pad_body(tt_ref, out_ref):
    blk = tt_ref[...]
    out_ref[...] = jnp.pad(blk.T, ((0, 0), (0, EMBED_DIM)))


_transpose_pad = pl.pallas_call(
    _transpose_pad_body,
    grid=(_GRID,),
    in_specs=[pl.BlockSpec((EMBED_DIM, _BC), lambda i: (0, i))],
    out_specs=pl.BlockSpec((_BC, 2 * EMBED_DIM), lambda i: (i, 0)),
    out_shape=jax.ShapeDtypeStruct((N_CLASSES, 2 * EMBED_DIM), jnp.float32),
)


@functools.partial(
    pl.kernel,
    mesh=plsc.VectorSubcoreMesh(core_axis_name="c", subcore_axis_name="s"),
    out_type=jax.ShapeDtypeStruct((BATCH, 2 * EMBED_DIM), jnp.float32),
    scratch_types=[
        pltpu.VMEM((_NCHUNKS, _CHUNK), jnp.int32),
        pltpu.VMEM((_B_PER_W, 2 * EMBED_DIM), jnp.float32),
        pltpu.SemaphoreType.DMA,
    ],
)
def _embed_lookup(idx_hbm, tablep_hbm, out_hbm, idx_v, rows_v, sem):
    wid = lax.axis_index("s") * _NC + lax.axis_index("c")
    base = wid * _B_PER_W
    pltpu.sync_copy(idx_hbm.at[wid], idx_v)
    copies = []
    for j in range(_NCHUNKS):
        copies.append(
            pltpu.async_copy(
                tablep_hbm.at[idx_v.at[j]],
                rows_v.at[pl.ds(j * _CHUNK, _CHUNK)],
                sem,
            )
        )
    for c in copies:
        c.wait()
    pltpu.sync_copy(rows_v, out_hbm.at[pl.ds(base, _B_PER_W)])


def kernel(class_ids, table):
    idx = class_ids.astype(jnp.int32).reshape(_NW, _NCHUNKS, _CHUNK)
    table_p = _transpose_pad(table.T)
    out = _embed_lookup(idx, table_p)
    return out[:, :EMBED_DIM].reshape(BATCH, 1, EMBED_DIM)
